# baseline (speedup 1.0000x reference)
import jax
import jax.numpy as jnp
from jax import lax
from jax.experimental import pallas as pl
from jax.experimental.pallas import tpu as pltpu
from jax.experimental.pallas import tpu_sc as plsc

_NUM_ENTITIES = 100000
_NUM_RELATIONS = 512
_EMB_DIM = 64
_NC = 2
_NS = 16
_NW = _NC * _NS
_SC_ROWS = 76800
_CHUNK = 400
_NFULL = _SC_ROWS // _CHUNK
_ROUNDS = _NFULL // _NW
_REL_PER_W = _NUM_RELATIONS // _NW
_TC_ROWS = _NUM_ENTITIES - _SC_ROWS
_TC_BLOCK = 800


def _sc_copy_body(ent_in, rel_in, ent_out, rel_out,
                  buf0, buf1, rbuf, sin0, sin1, sout0, sout1):
    wid = lax.axis_index("s") * _NC + lax.axis_index("c")
    bufs = (buf0, buf1)
    in_sems = (sin0, sin1)
    out_sems = (sout0, sout1)

    def rows(k):
        return pl.ds((wid + k * _NW) * _CHUNK, _CHUNK)

    in_copies = [None] * _ROUNDS
    out_copies = [None] * _ROUNDS
    in_copies[0] = pltpu.make_async_copy(ent_in.at[rows(0)], bufs[0],
                                         in_sems[0])
    in_copies[0].start()
    for k in range(_ROUNDS):
        b, nb = k % 2, (k + 1) % 2
        if k + 1 < _ROUNDS:
            if k >= 1:
                out_copies[k - 1].wait()
            in_copies[k + 1] = pltpu.make_async_copy(
                ent_in.at[rows(k + 1)], bufs[nb], in_sems[nb])
            in_copies[k + 1].start()
        in_copies[k].wait()
        out_copies[k] = pltpu.make_async_copy(bufs[b], ent_out.at[rows(k)],
                                              out_sems[b])
        out_copies[k].start()

    rrows = pl.ds(wid * _REL_PER_W, _REL_PER_W)
    pltpu.sync_copy(rel_in.at[rrows], rbuf)
    pltpu.sync_copy(rbuf, rel_out.at[rrows])

    out_copies[_ROUNDS - 2].wait()
    out_copies[_ROUNDS - 1].wait()


def _sc_copy(entity_emb, rel_emb):
    mesh = plsc.VectorSubcoreMesh(core_axis_name="c", subcore_axis_name="s")
    k = pl.kernel(
        _sc_copy_body,
        out_type=[
            jax.ShapeDtypeStruct((_NUM_ENTITIES, _EMB_DIM), jnp.float32),
            jax.ShapeDtypeStruct((_NUM_RELATIONS, _EMB_DIM), jnp.float32),
        ],
        mesh=mesh,
        scratch_types=[
            pltpu.VMEM((_CHUNK, _EMB_DIM), jnp.float32),
            pltpu.VMEM((_CHUNK, _EMB_DIM), jnp.float32),
            pltpu.VMEM((_REL_PER_W, _EMB_DIM), jnp.float32),
            pltpu.SemaphoreType.DMA,
            pltpu.SemaphoreType.DMA,
            pltpu.SemaphoreType.DMA,
            pltpu.SemaphoreType.DMA,
        ],
    )
    return k(entity_emb, rel_emb)


def _tc_body(x_ref, o_ref):
    o_ref[...] = x_ref[...]


def _tc_tail_copy(entity_emb):
    return pl.pallas_call(
        _tc_body,
        grid=(_TC_ROWS // _TC_BLOCK,),
        in_specs=[pl.BlockSpec((_TC_BLOCK, _EMB_DIM),
                               lambda i: (i + _SC_ROWS // _TC_BLOCK, 0))],
        out_specs=pl.BlockSpec((_TC_BLOCK, _EMB_DIM), lambda i: (i, 0)),
        out_shape=jax.ShapeDtypeStruct((_TC_ROWS, _EMB_DIM), jnp.float32),
    )(entity_emb)


def kernel(x_dict, edge_index, entity_emb, rel_emb):
    ent_sc, rel_out = _sc_copy(entity_emb, rel_emb)
    tc_part = _tc_tail_copy(entity_emb)
    entity_out = lax.dynamic_update_slice(ent_sc, tc_part, (_SC_ROWS, 0))
    return (entity_out, rel_out)
